# Initial kernel scaffold; baseline (speedup 1.0000x reference)
import jax
import jax.numpy as jnp
from jax.experimental import pallas as pl


def kernel(pointcloud, params):
    raise NotImplementedError("write your pallas kernel here")


# speedup vs baseline: 5.1500x; 5.1500x over previous
import functools

import jax
import jax.numpy as jnp
from jax import lax
from jax.experimental import pallas as pl

_B = 8
_F32 = jnp.float32
_PADC = 100.0


def _pad_rows(x, n, val):
    if x.shape[1] == n:
        return x
    pad = jnp.full((x.shape[0], n - x.shape[1]) + x.shape[2:], val, x.dtype)
    return jnp.concatenate([x, pad], axis=1)


def _fps_body(npoint, n, x_ref, y_ref, z_ref, o_ref):
    X = x_ref[...]
    Y = y_ref[...]
    Z = z_ref[...]
    iota = lax.broadcasted_iota(jnp.int32, (_B, n), 1)

    def step(t, carry):
        dists, far = carry
        oh = iota == far
        cx = jnp.sum(jnp.where(oh, X, 0.0), axis=1, keepdims=True)
        cy = jnp.sum(jnp.where(oh, Y, 0.0), axis=1, keepdims=True)
        cz = jnp.sum(jnp.where(oh, Z, 0.0), axis=1, keepdims=True)
        row = jnp.concatenate(
            [jnp.reshape(cx, (1, _B)), jnp.reshape(cy, (1, _B)),
             jnp.reshape(cz, (1, _B))], axis=1)
        o_ref[pl.ds(t, 1), :] = row
        d = (X - cx) ** 2 + (Y - cy) ** 2 + (Z - cz) ** 2
        dists = jnp.minimum(dists, d)
        far = jnp.argmax(dists, axis=1).astype(jnp.int32)[:, None]
        return dists, far

    lax.fori_loop(
        0, npoint, step,
        (jnp.full((_B, n), 1e10, _F32), jnp.zeros((_B, 1), jnp.int32)))


def _fps(xyz, npoint):
    n = xyz.shape[1]
    np_ = -(-n // 128) * 128
    if np_ != n:
        xyz = jnp.concatenate(
            [xyz, jnp.broadcast_to(xyz[:, :1, :], (_B, np_ - n, 3))], axis=1)
    c = pl.pallas_call(
        functools.partial(_fps_body, npoint, np_),
        out_shape=jax.ShapeDtypeStruct((npoint, 3 * _B), _F32),
    )(xyz[:, :, 0], xyz[:, :, 1], xyz[:, :, 2])
    return jnp.stack(
        [c[:, :_B].T, c[:, _B:2 * _B].T, c[:, 2 * _B:].T], axis=-1)


def _sa_body(sblk, np_, nreal, scales, c8_ref, xT_ref, pf_ref, *rest):
    nb = np_ // 128
    wA, wB = rest[0:7], rest[7:14]
    o_refs = rest[14:16]
    c8 = c8_ref[0]
    xT = xT_ref[0]
    pf = pf_ref[0]
    sa = jnp.sum(c8 * c8, axis=1, keepdims=True)
    sb = jnp.sum(xT * xT, axis=0, keepdims=True)
    d2 = sa + sb - 2.0 * jnp.dot(c8.astype(jnp.bfloat16),
                                 xT.astype(jnp.bfloat16),
                                 preferred_element_type=_F32)

    i128r = lax.broadcasted_iota(jnp.int32, (128, 128), 0)
    i128c = lax.broadcasted_iota(jnp.int32, (128, 128), 1)
    u128 = jnp.where(i128r <= i128c, 1.0, 0.0)

    cin = pf_ref.shape[-1]
    for (r2, ns, c1, c2, c3), w, o_ref in zip(scales, (wA, wB), o_refs):
        w1, wx, b1, w2, b2, w3, b3 = [r[...] for r in w]
        mfs = []
        granks = []
        run = jnp.zeros((sblk, 1), _F32)
        for t in range(nb):
            mf_t = jnp.where(d2[:, t * 128:(t + 1) * 128] <= r2, 1.0, 0.0)
            lr_t = jnp.dot(mf_t, u128, preferred_element_type=_F32,
                           precision=lax.Precision.HIGHEST)
            granks.append(lr_t + run)
            run = run + lr_t[:, 127:128]
            mfs.append(mf_t)
        total = run
        kr = lax.broadcasted_iota(jnp.int32, (1, ns), 1).astype(_F32) + 1.0
        tgt = jnp.where(kr <= total, kr, 1.0)
        g = jnp.zeros((sblk * ns, cin), _F32)
        for t in range(nb):
            sel_t = jnp.where(
                (granks[t][:, None, :] == tgt[:, :, None])
                & (mfs[t][:, None, :] > 0.0), 1.0, 0.0)
            g = g + jnp.dot(jnp.reshape(sel_t, (sblk * ns, 128)),
                            pf[t * 128:(t + 1) * 128, :],
                            preferred_element_type=_F32,
                            precision=lax.Precision.HIGHEST)
        totrep = jnp.reshape(
            jnp.broadcast_to(total[:, None, :], (sblk, ns, 1)),
            (sblk * ns, 1))
        g = jnp.where(totrep > 0.0, g, pf[nreal - 1:nreal, :])
        cpad = jnp.concatenate(
            [c8[:, :3], jnp.zeros((sblk, cin - 3), _F32)], axis=1)
        crep = jnp.reshape(
            jnp.broadcast_to(cpad[:, None, :], (sblk, ns, cin)),
            (sblk * ns, cin))
        g = g - crep
        h = jnp.maximum(jnp.dot(g, w1, preferred_element_type=_F32) + b1, 0.0)
        h = jnp.maximum(jnp.dot(h, w2, preferred_element_type=_F32) + b2, 0.0)
        h = jnp.maximum(jnp.dot(h, w3, preferred_element_type=_F32) + b3, 0.0)
        o_ref[0] = jnp.max(jnp.reshape(h, (sblk, ns, c3)), axis=1)


def _scale_weights(layers):
    w1 = layers[0]['W']
    w1t = w1.T
    wx8 = jnp.concatenate(
        [w1[:, :3].T, jnp.zeros((5, w1.shape[0]), _F32)], axis=0)
    out = [w1t, wx8, layers[0]['b'][None, :]]
    for l in layers[1:]:
        out.append(l['W'].T)
        out.append(l['b'][None, :])
    return out


def _sa_level(xyz, feats, new_xyz, layers2, radii, nsamples, sblk, sp, np_):
    s = new_xyz.shape[1]
    n = xyz.shape[1]
    cin = 3 + feats.shape[2]
    c8 = jnp.concatenate(
        [new_xyz, jnp.zeros((_B, s, 5), _F32)], axis=2)
    c8 = _pad_rows(c8, sp, _PADC)
    xp = _pad_rows(xyz, np_, _PADC)
    xT = jnp.concatenate(
        [jnp.transpose(xp, (0, 2, 1)), jnp.zeros((_B, 5, np_), _F32)], axis=1)
    pf = _pad_rows(jnp.concatenate([xp, _pad_rows(feats, np_, 0.0)], axis=2),
                   np_, 0.0)

    scales = []
    wflat = []
    wspecs = []
    for layers, r, ns in zip(layers2, radii, nsamples):
        dims = [l['W'].shape[0] for l in layers]
        scales.append((r * r, ns, dims[0], dims[1], dims[2]))
        ws = _scale_weights(layers)
        wflat += ws
        wspecs += [pl.BlockSpec(w.shape, lambda b, sidx: (0,) * w.ndim)
                   for w in ws]

    out_shapes = [jax.ShapeDtypeStruct((_B, sp, sc[4]), _F32) for sc in scales]
    out_specs = [pl.BlockSpec((1, sblk, sc[4]), lambda b, sidx: (b, sidx, 0))
                 for sc in scales]
    outs = pl.pallas_call(
        functools.partial(_sa_body, sblk, np_, n, scales),
        grid=(_B, sp // sblk),
        in_specs=[
            pl.BlockSpec((1, sblk, 8), lambda b, sidx: (b, sidx, 0)),
            pl.BlockSpec((1, 8, np_), lambda b, sidx: (b, 0, 0)),
            pl.BlockSpec((1, np_, cin), lambda b, sidx: (b, 0, 0)),
        ] + wspecs,
        out_specs=out_specs,
        out_shape=out_shapes,
    )(c8, xT, pf, *wflat)
    return jnp.concatenate([o[:, :s, :] for o in outs], axis=-1)


def _fp_body(nb1, n2p, ch, co, x1_ref, x2T_ref, f2_ref, f1_ref,
             w1_ref, b1_ref, w2_ref, b2_ref, o_ref):
    a8 = x1_ref[0]
    bT = x2T_ref[0]
    f2 = f2_ref[0]
    f1 = f1_ref[0]
    sa = jnp.sum(a8 * a8, axis=1, keepdims=True)
    sb = jnp.sum(bT * bT, axis=0, keepdims=True)
    d2 = sa + sb - 2.0 * jnp.dot(a8, bT, preferred_element_type=_F32)
    iota = lax.broadcasted_iota(jnp.int32, (nb1, n2p), 1)

    cur = d2
    ws = []
    rows = []
    for _ in range(3):
        m = jnp.min(cur, axis=1, keepdims=True)
        j = jnp.min(jnp.where(cur == m, iota, n2p), axis=1, keepdims=True)
        oh = jnp.where(iota == j, 1.0, 0.0)
        rows.append(jnp.dot(oh, f2, preferred_element_type=_F32,
                            precision=lax.Precision.HIGHEST))
        ws.append(1.0 / (m + 1e-8))
        cur = jnp.where(iota == j, 1e30, cur)
    wsum = ws[0] + ws[1] + ws[2]
    interp = (rows[0] * (ws[0] / wsum) + rows[1] * (ws[1] / wsum)
              + rows[2] * (ws[2] / wsum))
    new = jnp.concatenate([interp, f1], axis=1)
    h = jnp.maximum(
        jnp.dot(new, w1_ref[...], preferred_element_type=_F32)
        + b1_ref[...], 0.0)
    o_ref[0] = jnp.maximum(
        jnp.dot(h, w2_ref[...], preferred_element_type=_F32) + b2_ref[...],
        0.0)


def _fp_level(xyz1, xyz2, feats1, feats2, layers, nb1, n1p, n2p):
    n1 = xyz1.shape[1]
    n2 = xyz2.shape[1]
    c2 = feats2.shape[2]
    c1f = feats1.shape[2]
    w1 = layers[0]['W']
    ch = w1.shape[0]
    w1t = w1.T
    b1 = layers[0]['b'][None, :]
    w2 = layers[1]['W'].T
    co = w2.shape[1]
    b2 = layers[1]['b'][None, :]

    x1 = _pad_rows(jnp.concatenate(
        [xyz1, jnp.zeros((_B, n1, 5), _F32)], axis=2), n1p, _PADC)
    x2p = _pad_rows(xyz2, n2p, _PADC)
    x2T = jnp.concatenate(
        [jnp.transpose(x2p, (0, 2, 1)), jnp.zeros((_B, 5, n2p), _F32)],
        axis=1)
    f2 = _pad_rows(feats2, n2p, 0.0)
    f1 = _pad_rows(feats1, n1p, 0.0)

    full = lambda w: pl.BlockSpec(w.shape, lambda b, i: (0,) * w.ndim)
    out = pl.pallas_call(
        functools.partial(_fp_body, nb1, n2p, ch, co),
        grid=(_B, n1p // nb1),
        in_specs=[
            pl.BlockSpec((1, nb1, 8), lambda b, i: (b, i, 0)),
            pl.BlockSpec((1, 8, n2p), lambda b, i: (b, 0, 0)),
            pl.BlockSpec((1, n2p, c2), lambda b, i: (b, 0, 0)),
            pl.BlockSpec((1, nb1, c1f), lambda b, i: (b, i, 0)),
            full(w1t), full(b1), full(w2), full(b2),
        ],
        out_specs=pl.BlockSpec((1, nb1, co), lambda b, i: (b, i, 0)),
        out_shape=jax.ShapeDtypeStruct((_B, n1p, co), _F32),
    )(x1, x2T, f2, f1, w1t, b1, w2, b2)
    return out[:, :n1, :]


_SA_PLAN = [
    (1000, (0.0175, 0.025), (16, 32), 16, 1008, 4096),
    (500, (0.025, 0.05), (16, 32), 32, 512, 1024),
    (250, (0.05, 0.1), (16, 32), 32, 256, 512),
    (125, (0.1, 0.2), (16, 32), 32, 128, 256),
]

_FP_PLAN = {
    -1: (256, 256, 128),
    -2: (512, 512, 256),
    -3: (512, 1024, 512),
    -4: (512, 4096, 1024),
}


def kernel(pointcloud, params):
    xyz = pointcloud[..., :3]
    feats = pointcloud[..., 3:]
    l_xyz = [xyz]
    l_f = [feats]
    for lvl, (npoint, radii, nsamples, sblk, sp, np_) in enumerate(_SA_PLAN):
        new_xyz = _fps(l_xyz[lvl], npoint)
        nf = _sa_level(l_xyz[lvl], l_f[lvl], new_xyz, params['sa'][lvl],
                       radii, nsamples, sblk, sp, np_)
        l_xyz.append(new_xyz)
        l_f.append(nf)
    for i in range(-1, -5, -1):
        nb1, n1p, n2p = _FP_PLAN[i]
        l_f[i - 1] = _fp_level(l_xyz[i - 1], l_xyz[i], l_f[i - 1],
                               l_f[i], params['fp'][i], nb1, n1p, n2p)
    return jnp.transpose(l_f[0], (0, 2, 1))
